# baseline (speedup 1.0000x reference)
import jax
import jax.numpy as jnp
from jax.experimental import pallas as pl

B = 8192
D = 1024
H = 256
E = 3
BM = 512


def _fused_kernel(cs_ref, na_ref, e0_ref, e1_ref, e2_ref,
                  w1ag_ref, w1b_ref, u_ref, c1_ref, w2_ref, b2_ref,
                  out_ref, wts_ref):
    cs = cs_ref[...]
    mu = jnp.mean(cs, axis=1, keepdims=True)
    xc = cs - mu
    var = jnp.mean(xc * xc, axis=1, keepdims=True)
    s = jax.lax.rsqrt(var + 1e-5)
    t = (jnp.dot(cs.astype(jnp.bfloat16), w1ag_ref[...],
                 preferred_element_type=jnp.float32) * s
         + jnp.dot(na_ref[...].astype(jnp.bfloat16), w1b_ref[...],
                   preferred_element_type=jnp.float32))
    h = t - (s * mu) * u_ref[...] + c1_ref[...]
    h = 0.5 * h * (1.0 + jax.lax.erf(h * 0.7071067811865476))
    logits = jnp.dot(h, w2_ref[...], preferred_element_type=jnp.float32) + b2_ref[...]
    m = jnp.max(logits, axis=1, keepdims=True)
    ex = jnp.exp(logits - m)
    w = ex / jnp.sum(ex, axis=1, keepdims=True)
    wts_ref[...] = w
    out_ref[...] = (w[:, 0:1] * e0_ref[...]
                    + w[:, 1:2] * e1_ref[...]
                    + w[:, 2:3] * e2_ref[...])


def kernel(current_state, neighbor_activity, expert_out_0, expert_out_1, expert_out_2, ln_gamma, ln_beta, W1, b1, W2, b2):
    w1a = W1[:D]
    w1ag = (w1a * ln_gamma[:, None]).astype(jnp.bfloat16)
    w1b = W1[D:].astype(jnp.bfloat16)
    u = (ln_gamma @ w1a).reshape(1, H)
    c1 = (ln_beta @ w1a + b1).reshape(1, H)
    b2r = b2.reshape(1, E)

    grid = (B // BM,)
    row = lambda i: (i, 0)
    rep = lambda i: (0, 0)
    out, wts = pl.pallas_call(
        _fused_kernel,
        grid=grid,
        in_specs=[
            pl.BlockSpec((BM, D), row),
            pl.BlockSpec((BM, D), row),
            pl.BlockSpec((BM, D), row),
            pl.BlockSpec((BM, D), row),
            pl.BlockSpec((BM, D), row),
            pl.BlockSpec((D, H), rep),
            pl.BlockSpec((D, H), rep),
            pl.BlockSpec((1, H), rep),
            pl.BlockSpec((1, H), rep),
            pl.BlockSpec((H, E), rep),
            pl.BlockSpec((1, E), rep),
        ],
        out_specs=[
            pl.BlockSpec((BM, D), row),
            pl.BlockSpec((BM, E), row),
        ],
        out_shape=[
            jax.ShapeDtypeStruct((B, D), jnp.float32),
            jax.ShapeDtypeStruct((B, E), jnp.float32),
        ],
    )(current_state, neighbor_activity, expert_out_0, expert_out_1,
      expert_out_2, w1ag, w1b, u, c1, W2, b2r)
    return out, wts
